# Initial kernel scaffold; baseline (speedup 1.0000x reference)
import jax
import jax.numpy as jnp
from jax.experimental import pallas as pl


def kernel(hour, weekday, device, platform, hour_table, weekday_table, device_table, platform_table, W, b, gamma, beta):
    raise NotImplementedError("write your pallas kernel here")


# speedup vs baseline: 10.8122x; 10.8122x over previous
import functools

import jax
import jax.numpy as jnp
from jax.experimental import pallas as pl
from jax.experimental.pallas import tpu as pltpu

B = 16384
D = 64
BB = 2048
NB = B // BB

OFF_H, OFF_W, OFF_D, OFF_P = 0, 24, 31, 41
N_H, N_W, N_D, N_P = 24, 7, 10, 5


def _encoder_block(h_ref, w_ref, d_ref, p_ref, tcat_ref, W_ref, b_ref,
                   g_ref, be_ref, out_ref):
    ri = jax.lax.broadcasted_iota(jnp.int32, (64, 64), 0)
    tcat = tcat_ref[...]
    fused = jnp.zeros((64, 64), jnp.float32)
    for f, (lo, hi) in enumerate(((OFF_H, OFF_W), (OFF_W, OFF_D),
                                  (OFF_D, OFF_P), (OFF_P, 46))):
        mask = (ri >= lo) & (ri < hi)
        part = jnp.where(mask, tcat, 0.0)
        fused = fused + jnp.dot(part, W_ref[pl.ds(64 * f, 64), :],
                                preferred_element_type=jnp.float32)

    h = h_ref[0]
    w = w_ref[0]
    d = d_ref[0]
    p = p_ref[0]
    ci = jax.lax.broadcasted_iota(jnp.int32, (64, BB), 0)
    mh = ((ci == h).astype(jnp.float32)
          + (ci == (w + OFF_W)).astype(jnp.float32)
          + (ci == (d + OFF_D)).astype(jnp.float32)
          + (ci == (p + OFF_P)).astype(jnp.float32))

    x = jax.lax.dot_general(mh, fused, (((0,), (0,)), ((), ())),
                            preferred_element_type=jnp.float32)
    x = x + b_ref[...]
    mu = jnp.mean(x, axis=1, keepdims=True)
    xc = x - mu
    var = jnp.mean(xc * xc, axis=1, keepdims=True)
    y = xc * jax.lax.rsqrt(var + 1e-5) * g_ref[...] + be_ref[...]
    out_ref[...] = jnp.maximum(y, 0.0)


@functools.partial(jax.jit, static_argnames=("interpret",))
def _encode(hour, weekday, device, platform, tcat, W, b, gamma, beta,
            interpret=False):
    idx3 = lambda a: a.reshape(NB, 1, BB)
    idx_spec = pl.BlockSpec((1, 1, BB), lambda i: (i, 0, 0))
    full = lambda shape: pl.BlockSpec(shape, lambda i: tuple(0 for _ in shape))
    return pl.pallas_call(
        _encoder_block,
        grid=(NB,),
        in_specs=[idx_spec, idx_spec, idx_spec, idx_spec,
                  full((64, 64)), full((256, 64)), full((1, 64)),
                  full((1, 64)), full((1, 64))],
        out_specs=pl.BlockSpec((BB, 64), lambda i: (i, 0)),
        out_shape=jax.ShapeDtypeStruct((B, D), jnp.float32),
        interpret=interpret,
    )(idx3(hour), idx3(weekday), idx3(device), idx3(platform),
      tcat, W, b.reshape(1, D), gamma.reshape(1, D), beta.reshape(1, D))


def kernel(hour, weekday, device, platform, hour_table, weekday_table,
           device_table, platform_table, W, b, gamma, beta):
    tcat = jnp.zeros((64, D), jnp.float32)
    tcat = jax.lax.dynamic_update_slice(tcat, hour_table, (OFF_H, 0))
    tcat = jax.lax.dynamic_update_slice(tcat, weekday_table, (OFF_W, 0))
    tcat = jax.lax.dynamic_update_slice(tcat, device_table, (OFF_D, 0))
    tcat = jax.lax.dynamic_update_slice(tcat, platform_table, (OFF_P, 0))
    return _encode(hour, weekday, device, platform, tcat, W, b, gamma, beta)
